# baseline (device time: 124624 ns/iter reference)
import functools

import jax
import jax.numpy as jnp
from jax import lax
from jax.experimental import pallas as pl
from jax.experimental.pallas import tpu as pltpu

N_DEV = 8

_MASK = {"x": 1, "y": 3, "z": 4}

_PARTS = (
    (0, 704, ("x", "y", "z")),
    (704, 704, ("x", "y", "z")),
    (1408, 704, ("y", "z", "x")),
    (2112, 640, ("y", "z", "x")),
    (2752, 704, ("z", "x", "y")),
    (3456, 640, ("z", "x", "y")),
)
_NP = len(_PARTS)


def kernel(x):
    m_per, n = x.shape
    assert m_per == 4096

    def body(x_ref, out_ref, comm_ref, rs_send, rs_recv, ag_send, ag_recv):
        me = lax.axis_index("i")
        partners = {d: me ^ mask for d, mask in _MASK.items()}
        coords = {
            "x": (me ^ (me >> 1)) & 1,
            "y": (me >> 1) & 1,
            "z": (me >> 2) & 1,
        }

        barrier_sem = pltpu.get_barrier_semaphore()
        for d in ("x", "y", "z"):
            pl.semaphore_signal(
                barrier_sem, inc=1,
                device_id=(partners[d],), device_id_type=pl.DeviceIdType.MESH,
            )
        pl.semaphore_wait(barrier_sem, 3)


        seg_lo = {}
        halves = {}

        def rs_descr(p, k):
            base, rows, dims = _PARTS[p]
            d = dims[k]
            part = partners[d]
            half = coords[d]
            halves[(p, k)] = half
            l2 = rows >> (k + 1)
            lo = seg_lo[p]
            send_lo = lo + (1 - half) * l2
            off = base + (0 if k == 0 else (rows >> 1) if k == 1 else 3 * (rows >> 2))
            src = x_ref if k == 0 else out_ref
            return pltpu.make_async_remote_copy(
                src_ref=src.at[pl.ds(send_lo, l2)],
                dst_ref=comm_ref.at[pl.ds(off, l2)],
                send_sem=rs_send.at[p, k],
                recv_sem=rs_recv.at[p, k],
                device_id=(part,),
                device_id_type=pl.DeviceIdType.MESH,
            ), off, l2

        def ag_descr(p, j, lo, l, d):
            return pltpu.make_async_remote_copy(
                src_ref=out_ref.at[pl.ds(lo, l)],
                dst_ref=out_ref.at[pl.ds(lo, l)],
                send_sem=ag_send.at[p, j],
                recv_sem=ag_recv.at[p, j],
                device_id=(partners[d],),
                device_id_type=pl.DeviceIdType.MESH,
            )

        for p in range(_NP):
            seg_lo[p] = jnp.int32(_PARTS[p][0])
        inflight = {}
        for p in range(_NP):
            rdma, off, l2 = rs_descr(p, 0)
            rdma.start()
            inflight[p] = (rdma, off, l2)

        _ORDER = (0, 2, 4, 1, 3, 5)
        ag_inflight = {}
        for k in range(3):
            for p in _ORDER:
                rdma, off, l2 = inflight[p]
                rdma.wait()
                half = halves[(p, k)]
                keep_lo = seg_lo[p] + half * l2
                acc_src = x_ref if k == 0 else out_ref
                seg_lo[p] = keep_lo
                if k < 2:
                    l4 = l2 >> 1
                    nxt_half = coords[_PARTS[p][2][k + 1]]
                    q1 = (1 - nxt_half) * l4
                    idx = pl.ds(keep_lo + q1, l4)
                    out_ref[idx] = acc_src[idx] + comm_ref[pl.ds(off + q1, l4)]
                    nxt, noff, nl2 = rs_descr(p, k + 1)
                    nxt.start()
                    inflight[p] = (nxt, noff, nl2)
                    q2 = l4 - q1
                    idx = pl.ds(keep_lo + q2, l4)
                    out_ref[idx] = acc_src[idx] + comm_ref[pl.ds(off + q2, l4)]
                else:
                    idx = pl.ds(keep_lo, l2)
                    out_ref[idx] = acc_src[idx] + comm_ref[pl.ds(off, l2)]
                    d0, d1, d2 = _PARTS[p][2]
                    lo3, l = keep_lo, l2
                    ag0 = ag_descr(p, 0, lo3, l, d2)
                    ag1a = ag_descr(p, 1, lo3, l, d1)
                    ag0.start()
                    ag1a.start()
                    ag_inflight[p] = (lo3, l, ag0, ag1a)

        stage_b = {}
        for p in _ORDER:
            lo3, l, ag0, ag1a = ag_inflight[p]
            d0, d1, d2 = _PARTS[p][2]
            ag0.wait()
            h2 = halves[(p, 2)]
            sib2_lo = lo3 + (1 - 2 * h2) * l
            lo1 = lo3 - h2 * l
            ag1b = ag_descr(p, 2, sib2_lo, l, d1)
            ag2a = ag_descr(p, 3, lo1, 2 * l, d0)
            ag1b.start()
            ag2a.start()
            stage_b[p] = (lo1, l, ag1a, ag1b, ag2a)

        stage_c = {}
        for p in _ORDER:
            lo1, l, ag1a, ag1b, ag2a = stage_b[p]
            d0 = _PARTS[p][2][0]
            ag1a.wait()
            ag1b.wait()
            h1 = halves[(p, 1)]
            sib1_lo = lo1 + (1 - 2 * h1) * 2 * l
            ag2b = ag_descr(p, 4, sib1_lo, 2 * l, d0)
            ag2b.start()
            stage_c[p] = (ag2a, ag2b)

        for p in _ORDER:
            ag2a, ag2b = stage_c[p]
            ag2a.wait()
            ag2b.wait()

        @functools.partial(
            pl.run_scoped, second_barrier=pltpu.SemaphoreType.REGULAR
        )
        def _(second_barrier):
            for d in ("x", "y", "z"):
                pl.semaphore_signal(
                    second_barrier, inc=1,
                    device_id=(partners[d],),
                    device_id_type=pl.DeviceIdType.MESH,
                )
            pl.semaphore_wait(second_barrier, 3)

    return pl.pallas_call(
        body,
        out_shape=jax.ShapeDtypeStruct((m_per, n), x.dtype),
        in_specs=[pl.BlockSpec(memory_space=pltpu.VMEM)],
        out_specs=pl.BlockSpec(memory_space=pltpu.VMEM),
        scratch_shapes=[
            pltpu.VMEM((m_per, n), x.dtype),
            pltpu.SemaphoreType.DMA((_NP, 3)),
            pltpu.SemaphoreType.DMA((_NP, 3)),
            pltpu.SemaphoreType.DMA((_NP, 5)),
            pltpu.SemaphoreType.DMA((_NP, 5)),
        ],
        compiler_params=pltpu.CompilerParams(collective_id=0),
    )(x)


# device time: 124413 ns/iter; 1.0017x vs baseline; 1.0017x over previous
import functools

import jax
import jax.numpy as jnp
from jax import lax
from jax.experimental import pallas as pl
from jax.experimental.pallas import tpu as pltpu

N_DEV = 8

_MASK = {"x": 1, "y": 3, "z": 4}

_PARTS = (
    (0, 704, ("x", "y", "z")),
    (704, 704, ("x", "y", "z")),
    (1408, 704, ("y", "z", "x")),
    (2112, 640, ("y", "z", "x")),
    (2752, 704, ("z", "x", "y")),
    (3456, 640, ("z", "x", "y")),
)
_NP = len(_PARTS)


def kernel(x):
    m_per, n = x.shape
    assert m_per == 4096

    def body(x_ref, out_ref, comm_ref, rs_send, rs_recv, ag_send, ag_recv):
        me = lax.axis_index("i")
        partners = {d: me ^ mask for d, mask in _MASK.items()}
        coords = {
            "x": (me ^ (me >> 1)) & 1,
            "y": (me >> 1) & 1,
            "z": (me >> 2) & 1,
        }

        barrier_sem = pltpu.get_barrier_semaphore()
        for d in ("x", "y", "z"):
            pl.semaphore_signal(
                barrier_sem, inc=1,
                device_id=(partners[d],), device_id_type=pl.DeviceIdType.MESH,
            )
        pl.semaphore_wait(barrier_sem, 3)


        seg_lo = {}
        halves = {}

        def rs_descr(p, k):
            base, rows, dims = _PARTS[p]
            d = dims[k]
            part = partners[d]
            half = coords[d]
            halves[(p, k)] = half
            l2 = rows >> (k + 1)
            lo = seg_lo[p]
            send_lo = lo + (1 - half) * l2
            off = base + (0 if k == 0 else (rows >> 1) if k == 1 else 3 * (rows >> 2))
            src = x_ref if k == 0 else out_ref
            return pltpu.make_async_remote_copy(
                src_ref=src.at[pl.ds(send_lo, l2)],
                dst_ref=comm_ref.at[pl.ds(off, l2)],
                send_sem=rs_send.at[p, k],
                recv_sem=rs_recv.at[p, k],
                device_id=(part,),
                device_id_type=pl.DeviceIdType.MESH,
            ), off, l2

        def ag_descr(p, k):
            base, rows, dims = _PARTS[p]
            d = dims[2 - k]
            part = partners[d]
            l = rows >> (3 - k)
            lo = seg_lo[p]
            return pltpu.make_async_remote_copy(
                src_ref=out_ref.at[pl.ds(lo, l)],
                dst_ref=out_ref.at[pl.ds(lo, l)],
                send_sem=ag_send.at[p, k],
                recv_sem=ag_recv.at[p, k],
                device_id=(part,),
                device_id_type=pl.DeviceIdType.MESH,
            )

        for p in range(_NP):
            seg_lo[p] = jnp.int32(_PARTS[p][0])
        inflight = {}
        for p in range(_NP):
            rdma, off, l2 = rs_descr(p, 0)
            rdma.start()
            inflight[p] = (rdma, off, l2)

        _ORDER = (0, 2, 4, 1, 3, 5)
        ag_inflight = {}
        for k in range(3):
            for p in _ORDER:
                rdma, off, l2 = inflight[p]
                rdma.wait()
                half = halves[(p, k)]
                keep_lo = seg_lo[p] + half * l2
                acc_src = x_ref if k == 0 else out_ref
                seg_lo[p] = keep_lo
                if k < 2:
                    l4 = l2 >> 1
                    nxt_half = coords[_PARTS[p][2][k + 1]]
                    q1 = (1 - nxt_half) * l4
                    idx = pl.ds(keep_lo + q1, l4)
                    out_ref[idx] = acc_src[idx] + comm_ref[pl.ds(off + q1, l4)]
                    nxt, noff, nl2 = rs_descr(p, k + 1)
                    nxt.start()
                    inflight[p] = (nxt, noff, nl2)
                    q2 = l4 - q1
                    idx = pl.ds(keep_lo + q2, l4)
                    out_ref[idx] = acc_src[idx] + comm_ref[pl.ds(off + q2, l4)]
                else:
                    idx = pl.ds(keep_lo, l2)
                    out_ref[idx] = acc_src[idx] + comm_ref[pl.ds(off, l2)]
                    ag = ag_descr(p, 0)
                    ag.start()
                    ag_inflight[p] = ag

        for k in range(3):
            for p in _ORDER:
                rows = _PARTS[p][1]
                l = rows >> (3 - k)
                ag_inflight[p].wait()
                seg_lo[p] = seg_lo[p] - halves[(p, 2 - k)] * l
                if k < 2:
                    ag = ag_descr(p, k + 1)
                    ag.start()
                    ag_inflight[p] = ag

        @functools.partial(
            pl.run_scoped, second_barrier=pltpu.SemaphoreType.REGULAR
        )
        def _(second_barrier):
            for d in ("x", "y", "z"):
                pl.semaphore_signal(
                    second_barrier, inc=1,
                    device_id=(partners[d],),
                    device_id_type=pl.DeviceIdType.MESH,
                )
            pl.semaphore_wait(second_barrier, 3)

    return pl.pallas_call(
        body,
        out_shape=jax.ShapeDtypeStruct((m_per, n), x.dtype),
        in_specs=[pl.BlockSpec(memory_space=pltpu.VMEM)],
        out_specs=pl.BlockSpec(memory_space=pltpu.VMEM),
        scratch_shapes=[
            pltpu.VMEM((m_per, n), x.dtype),
            pltpu.SemaphoreType.DMA((_NP, 3)),
            pltpu.SemaphoreType.DMA((_NP, 3)),
            pltpu.SemaphoreType.DMA((_NP, 3)),
            pltpu.SemaphoreType.DMA((_NP, 3)),
        ],
        compiler_params=pltpu.CompilerParams(collective_id=0),
    )(x)
